# baseline (speedup 1.0000x reference)
import functools

import jax
import jax.numpy as jnp
from jax import lax
from jax.experimental import pallas as pl
from jax.experimental.pallas import tpu as pltpu
from jax.experimental.pallas import tpu_sc as plsc

EMBED_DIM = 64
NUM_CORES = 2
BLK = 16384
NBUF = 2


def _emb_kernel(ids_hbm, table_hbm, out_hbm, stage, sems):
    cid = lax.axis_index("c")
    sid = lax.axis_index("s")
    n = out_hbm.shape[0]
    half = n // NUM_CORES
    nblocks = half // BLK
    base = cid * half

    @pl.when(sid == 0)
    def _():
        def body(i, carry):
            for b in range(NBUF):
                @pl.when(i > 0)
                def _(b=b):
                    pltpu.make_async_copy(
                        stage[b], out_hbm.at[pl.ds(0, BLK)], sems[b]).wait()
                pltpu.async_copy(
                    stage[b],
                    out_hbm.at[pl.ds(base + (i * NBUF + b) * BLK, BLK)],
                    sems[b])
            return carry

        lax.fori_loop(0, nblocks // NBUF, body, 0)
        for b in range(NBUF):
            pltpu.make_async_copy(
                stage[b], out_hbm.at[pl.ds(0, BLK)], sems[b]).wait()


def kernel(phoneme_ids, table):
    b, t = phoneme_ids.shape
    n = b * t
    ids2d = phoneme_ids.reshape(n // 128, 128).astype(jnp.int32)

    emb = functools.partial(
        pl.kernel,
        mesh=plsc.VectorSubcoreMesh(core_axis_name="c", subcore_axis_name="s"),
        out_type=jax.ShapeDtypeStruct((n, EMBED_DIM), jnp.float32),
        scratch_types=[
            [pltpu.VMEM_SHARED((BLK, EMBED_DIM), jnp.float32)
             for _ in range(NBUF)],
            [pltpu.SemaphoreType.DMA for _ in range(NBUF)],
        ],
        compiler_params=pltpu.CompilerParams(use_tc_tiling_on_sc=False),
    )(_emb_kernel)

    out = emb(ids2d, table)
    return out.reshape(b, t, EMBED_DIM)


# speedup vs baseline: 1.0445x; 1.0445x over previous
import functools

import jax
import jax.numpy as jnp
from jax import lax
from jax.experimental import pallas as pl
from jax.experimental.pallas import tpu as pltpu
from jax.experimental.pallas import tpu_sc as plsc

EMBED_DIM = 64
NUM_CORES = 2
NUM_SUBCORES = 16
NUM_WORKERS = NUM_CORES * NUM_SUBCORES
CHUNK = 40
CHUNKS_PER_SEQ = 5
SEQS_PER_STEP = 4
GATHERS_PER_STEP = CHUNKS_PER_SEQ * SEQS_PER_STEP


def _emb_kernel(ids_hbm, table_hbm, out_hbm,
                idx0, idx1, rows0, rows1, table_sh,
                semg0, semg1, semw0, semw1, semi0, semi1):
    wid = lax.axis_index("s") * NUM_CORES + lax.axis_index("c")
    n_seq, seq_len = ids_hbm.shape
    seqs_per_worker = n_seq // NUM_WORKERS
    steps = seqs_per_worker // SEQS_PER_STEP
    nouter = steps // 2
    seq_base = wid * seqs_per_worker

    @pl.when(lax.axis_index("s") == 0)
    def _():
        pltpu.sync_copy(table_hbm, table_sh)
    plsc.subcore_barrier()

    def start_ids(g, idx_v, sem):
        s0 = seq_base + g * SEQS_PER_STEP
        pltpu.async_copy(ids_hbm.at[pl.ds(s0, SEQS_PER_STEP)], idx_v, sem)

    def wait_ids(idx_v, sem):
        pltpu.make_async_copy(ids_hbm.at[pl.ds(0, SEQS_PER_STEP)],
                              idx_v, sem).wait()

    def start_gathers(idx_v, rows_v, sem):
        for j in range(GATHERS_PER_STEP):
            s, h = divmod(j, CHUNKS_PER_SEQ)
            pltpu.async_copy(
                table_sh.at[idx_v.at[s, pl.ds(h * CHUNK, CHUNK)]],
                rows_v.at[s, pl.ds(h * CHUNK, CHUNK)],
                sem,
            )

    def wait_gathers(rows_v, sem):
        pltpu.make_async_copy(out_hbm.at[pl.ds(0, SEQS_PER_STEP)],
                              rows_v, sem).wait()

    def start_write(g, rows_v, sem):
        s0 = seq_base + g * SEQS_PER_STEP
        pltpu.async_copy(rows_v, out_hbm.at[pl.ds(s0, SEQS_PER_STEP)], sem)

    def wait_write(rows_v, sem):
        pltpu.make_async_copy(rows_v, out_hbm.at[pl.ds(0, SEQS_PER_STEP)],
                              sem).wait()

    start_ids(0, idx0, semi0)
    wait_ids(idx0, semi0)
    start_gathers(idx0, rows0, semg0)
    start_ids(1, idx1, semi1)

    def body(i, carry):
        g0 = 2 * i

        @pl.when(i > 0)
        def _():
            wait_write(rows1, semw1)
        wait_ids(idx1, semi1)
        start_gathers(idx1, rows1, semg1)
        wait_gathers(rows0, semg0)

        @pl.when(g0 + 2 < steps)
        def _():
            start_ids(g0 + 2, idx0, semi0)
        start_write(g0, rows0, semw0)

        @pl.when(g0 + 2 < steps)
        def _():
            wait_write(rows0, semw0)
            wait_ids(idx0, semi0)
            start_gathers(idx0, rows0, semg0)
        wait_gathers(rows1, semg1)

        @pl.when(g0 + 3 < steps)
        def _():
            start_ids(g0 + 3, idx1, semi1)
        start_write(g0 + 1, rows1, semw1)
        return carry

    lax.fori_loop(0, nouter, body, 0)
    wait_write(rows0, semw0)
    wait_write(rows1, semw1)


def kernel(phoneme_ids, table):
    b, t = phoneme_ids.shape
    ids = phoneme_ids.astype(jnp.int32)

    emb = functools.partial(
        pl.kernel,
        mesh=plsc.VectorSubcoreMesh(core_axis_name="c", subcore_axis_name="s"),
        out_type=jax.ShapeDtypeStruct((b, t, EMBED_DIM), jnp.float32),
        scratch_types=[
            pltpu.VMEM((SEQS_PER_STEP, 200), jnp.int32),
            pltpu.VMEM((SEQS_PER_STEP, 200), jnp.int32),
            pltpu.VMEM((SEQS_PER_STEP, 200, EMBED_DIM), jnp.float32),
            pltpu.VMEM((SEQS_PER_STEP, 200, EMBED_DIM), jnp.float32),
            pltpu.VMEM_SHARED((1000, EMBED_DIM), jnp.float32),
            pltpu.SemaphoreType.DMA,
            pltpu.SemaphoreType.DMA,
            pltpu.SemaphoreType.DMA,
            pltpu.SemaphoreType.DMA,
            pltpu.SemaphoreType.DMA,
            pltpu.SemaphoreType.DMA,
        ],
        compiler_params=pltpu.CompilerParams(use_tc_tiling_on_sc=False),
    )(_emb_kernel)

    return emb(ids, table)


# speedup vs baseline: 1.3543x; 1.2966x over previous
import functools

import jax
import jax.numpy as jnp
from jax import lax
from jax.experimental import pallas as pl
from jax.experimental.pallas import tpu as pltpu
from jax.experimental.pallas import tpu_sc as plsc

EMBED_DIM = 64
VOCAB_PAD = 1024
NUM_CORES = 2
NUM_SUBCORES = 16
NUM_WORKERS = NUM_CORES * NUM_SUBCORES
T_PER_GROUP = 8
B_BLOCK = 128
LANES = 16


def _emb_kernel(ids_t_hbm, table_hbm, out_hbm,
                table_v, idx0, idx1, tr0, tr1,
                semi0, semi1, semw0, semw1):
    wid = lax.axis_index("s") * NUM_CORES + lax.axis_index("c")
    n_t, n_b = ids_t_hbm.shape
    n_tg = n_t // T_PER_GROUP
    n_bb = n_b // B_BLOCK
    groups = (n_tg * n_bb) // NUM_WORKERS
    g_base = wid * groups

    pltpu.sync_copy(table_hbm, table_v)

    def ids_start(k, idxv, sem):
        g_lin = g_base + k
        tg = g_lin // n_bb
        bb = g_lin % n_bb
        r0 = pl.multiple_of(tg * T_PER_GROUP, T_PER_GROUP)
        c0 = pl.multiple_of(bb * B_BLOCK, B_BLOCK)
        pltpu.async_copy(
            ids_t_hbm.at[pl.ds(r0, T_PER_GROUP), pl.ds(c0, B_BLOCK)],
            idxv, sem)

    def ids_wait(idxv, sem):
        pltpu.make_async_copy(
            ids_t_hbm.at[pl.ds(0, T_PER_GROUP), pl.ds(0, B_BLOCK)],
            idxv, sem).wait()

    def wait_write(tr, sem):
        pltpu.make_async_copy(tr, out_hbm.at[0, :, pl.ds(0, B_BLOCK)],
                              sem).wait()

    trs = (tr0, tr1)
    semws = (semw0, semw1)

    def do_group(k, idxv):
        g_lin = g_base + k
        tg = g_lin // n_bb
        bb = g_lin % n_bb
        c0 = pl.multiple_of(bb * B_BLOCK, B_BLOCK)
        for tt in range(T_PER_GROUP):
            tr = trs[tt % 2]
            semw = semws[tt % 2]

            @pl.when(k * T_PER_GROUP + tt >= 2)
            def _():
                wait_write(tr, semw)

            ids_g = [idxv[tt, pl.ds(g * LANES, LANES)]
                     for g in range(B_BLOCK // LANES)]

            def dbody(d, carry):
                base = d * VOCAB_PAD
                for g in range(B_BLOCK // LANES):
                    v = plsc.load_gather(table_v, [ids_g[g] + base])
                    tr[d, pl.ds(g * LANES, LANES)] = v
                return carry

            lax.fori_loop(0, EMBED_DIM, dbody, 0)
            t = tg * T_PER_GROUP + tt
            pltpu.async_copy(tr, out_hbm.at[t, :, pl.ds(c0, B_BLOCK)], semw)

    ids_start(0, idx0, semi0)
    ids_start(1, idx1, semi1)

    def outer(r, carry):
        k0 = 2 * r
        ids_wait(idx0, semi0)
        do_group(k0, idx0)

        @pl.when(k0 + 2 < groups)
        def _():
            ids_start(k0 + 2, idx0, semi0)
        ids_wait(idx1, semi1)
        do_group(k0 + 1, idx1)

        @pl.when(k0 + 3 < groups)
        def _():
            ids_start(k0 + 3, idx1, semi1)
        return carry

    lax.fori_loop(0, groups // 2, outer, 0)
    wait_write(tr0, semw0)
    wait_write(tr1, semw1)


def kernel(phoneme_ids, table):
    b, t = phoneme_ids.shape
    ids_t = jnp.transpose(phoneme_ids.astype(jnp.int32))
    table_t = jnp.zeros((EMBED_DIM, VOCAB_PAD), jnp.float32)
    table_t = table_t.at[:, :table.shape[0]].set(jnp.transpose(table))
    table_flat = table_t.reshape(-1)

    emb = functools.partial(
        pl.kernel,
        mesh=plsc.VectorSubcoreMesh(core_axis_name="c", subcore_axis_name="s"),
        out_type=jax.ShapeDtypeStruct((t, EMBED_DIM, b), jnp.float32),
        scratch_types=[
            pltpu.VMEM((EMBED_DIM * VOCAB_PAD,), jnp.float32),
            pltpu.VMEM((T_PER_GROUP, B_BLOCK), jnp.int32),
            pltpu.VMEM((T_PER_GROUP, B_BLOCK), jnp.int32),
            pltpu.VMEM((EMBED_DIM, B_BLOCK), jnp.float32),
            pltpu.VMEM((EMBED_DIM, B_BLOCK), jnp.float32),
            pltpu.SemaphoreType.DMA,
            pltpu.SemaphoreType.DMA,
            pltpu.SemaphoreType.DMA,
            pltpu.SemaphoreType.DMA,
        ],
        compiler_params=pltpu.CompilerParams(use_tc_tiling_on_sc=True, needs_layout_passes=False),
    )(_emb_kernel)

    out_tdb = emb(ids_t, table_flat)
    return jnp.transpose(out_tdb, (2, 0, 1))


# speedup vs baseline: 5.1904x; 3.8324x over previous
import functools

import jax
import jax.numpy as jnp
from jax import lax
from jax.experimental import pallas as pl
from jax.experimental.pallas import tpu as pltpu
from jax.experimental.pallas import tpu_sc as plsc

EMBED_DIM = 64
VOCAB_PAD = 1024
NUM_CORES = 2
NUM_SUBCORES = 16
NUM_WORKERS = NUM_CORES * NUM_SUBCORES
T_PER_GROUP = 8
B_BLOCK = 128
LANES = 16


def _emb_kernel(ids_t_hbm, table_hbm, out_hbm,
                table_v, idx0, idx1, tr0, tr1,
                semi0, semi1, semw0, semw1):
    wid = lax.axis_index("s") * NUM_CORES + lax.axis_index("c")
    n_t, n_b = ids_t_hbm.shape
    n_tg = n_t // T_PER_GROUP
    n_bb = n_b // B_BLOCK
    groups = (n_tg * n_bb) // NUM_WORKERS
    g_base = wid * groups

    pltpu.sync_copy(table_hbm, table_v)

    def ids_start(k, idxv, sem):
        g_lin = g_base + k
        tg = g_lin // n_bb
        bb = g_lin % n_bb
        r0 = pl.multiple_of(tg * T_PER_GROUP, T_PER_GROUP)
        c0 = pl.multiple_of(bb * B_BLOCK, B_BLOCK)
        pltpu.async_copy(
            ids_t_hbm.at[pl.ds(r0, T_PER_GROUP), pl.ds(c0, B_BLOCK)],
            idxv, sem)

    def ids_wait(idxv, sem):
        pltpu.make_async_copy(
            ids_t_hbm.at[pl.ds(0, T_PER_GROUP), pl.ds(0, B_BLOCK)],
            idxv, sem).wait()

    def wait_write(tr, sem):
        pltpu.make_async_copy(tr, out_hbm.at[0, :, pl.ds(0, B_BLOCK)],
                              sem).wait()

    trs = (tr0, tr1)
    semws = (semw0, semw1)

    def do_group(k, idxv):
        g_lin = g_base + k
        tg = g_lin // n_bb
        bb = g_lin % n_bb
        c0 = pl.multiple_of(bb * B_BLOCK, B_BLOCK)
        for tt in range(T_PER_GROUP):
            tr = trs[tt % 2]
            semw = semws[tt % 2]

            @pl.when(k * T_PER_GROUP + tt >= 2)
            def _():
                wait_write(tr, semw)

            ids_g = [idxv[tt, pl.ds(g * LANES, LANES)]
                     for g in range(B_BLOCK // LANES)]

            @plsc.parallel_loop(0, EMBED_DIM, unroll=4)
            def dbody(d):
                base = d * VOCAB_PAD
                for g in range(B_BLOCK // LANES):
                    v = plsc.load_gather(table_v, [ids_g[g] + base])
                    tr[d, pl.ds(g * LANES, LANES)] = v
            t = tg * T_PER_GROUP + tt
            pltpu.async_copy(tr, out_hbm.at[t, :, pl.ds(c0, B_BLOCK)], semw)

    ids_start(0, idx0, semi0)
    ids_start(1, idx1, semi1)

    def outer(r, carry):
        k0 = 2 * r
        ids_wait(idx0, semi0)
        do_group(k0, idx0)

        @pl.when(k0 + 2 < groups)
        def _():
            ids_start(k0 + 2, idx0, semi0)
        ids_wait(idx1, semi1)
        do_group(k0 + 1, idx1)

        @pl.when(k0 + 3 < groups)
        def _():
            ids_start(k0 + 3, idx1, semi1)
        return carry

    lax.fori_loop(0, groups // 2, outer, 0)
    wait_write(tr0, semw0)
    wait_write(tr1, semw1)


def kernel(phoneme_ids, table):
    b, t = phoneme_ids.shape
    ids_t = jnp.transpose(phoneme_ids.astype(jnp.int32))
    table_t = jnp.zeros((EMBED_DIM, VOCAB_PAD), jnp.float32)
    table_t = table_t.at[:, :table.shape[0]].set(jnp.transpose(table))
    table_flat = table_t.reshape(-1)

    emb = functools.partial(
        pl.kernel,
        mesh=plsc.VectorSubcoreMesh(core_axis_name="c", subcore_axis_name="s"),
        out_type=jax.ShapeDtypeStruct((t, EMBED_DIM, b), jnp.float32),
        scratch_types=[
            pltpu.VMEM((EMBED_DIM * VOCAB_PAD,), jnp.float32),
            pltpu.VMEM((T_PER_GROUP, B_BLOCK), jnp.int32),
            pltpu.VMEM((T_PER_GROUP, B_BLOCK), jnp.int32),
            pltpu.VMEM((EMBED_DIM, B_BLOCK), jnp.float32),
            pltpu.VMEM((EMBED_DIM, B_BLOCK), jnp.float32),
            pltpu.SemaphoreType.DMA,
            pltpu.SemaphoreType.DMA,
            pltpu.SemaphoreType.DMA,
            pltpu.SemaphoreType.DMA,
        ],
        compiler_params=pltpu.CompilerParams(use_tc_tiling_on_sc=True, needs_layout_passes=False),
    )(_emb_kernel)

    out_tdb = emb(ids_t, table_flat)
    return jnp.transpose(out_tdb, (2, 0, 1))
